# baseline (speedup 1.0000x reference)
import functools

import jax
import jax.numpy as jnp
from jax import lax
from jax.experimental import pallas as pl
from jax.experimental.pallas import tpu as pltpu
from jax.experimental.pallas import tpu_sc as plsc

V = 1_000_000
D = 64
B = 327_680

HB = 2048
GRID = -(-V // (2 * HB))
NP = GRID * HB
MAXBLK = -(-V // HB) - 1


def _pack_kernel(a_ref, b_ref, out_ref):
    out_ref[...] = jnp.concatenate(
        [jnp.transpose(a_ref[...], (1, 0)),
         jnp.transpose(b_ref[...], (1, 0))], axis=1)


def _pack(table_t):
    return pl.pallas_call(
        _pack_kernel,
        grid=(GRID,),
        in_specs=[
            pl.BlockSpec((D, HB), lambda g: (0, jnp.minimum(2 * g, MAXBLK))),
            pl.BlockSpec((D, HB), lambda g: (0, jnp.minimum(2 * g + 1, MAXBLK))),
        ],
        out_specs=pl.BlockSpec((HB, 128), lambda g: (g, 0)),
        out_shape=jax.ShapeDtypeStruct((NP, 128), jnp.float32),
    )(table_t, table_t)


NC, NS = 2, 16
NW = NC * NS
CHUNK = 128
PER_W = B // NW
N_CHUNKS = PER_W // CHUNK
NBUF = 5
N_ROUNDS = N_CHUNKS // NBUF


TCB = B // 128


def _make_gather():
    mesh = plsc.VectorSubcoreMesh(
        core_axis_name="c", subcore_axis_name="s",
        num_cores=NC, num_subcores=NS)

    @functools.partial(
        pl.kernel,
        out_type=jax.ShapeDtypeStruct((8, TCB, 8, 128), jnp.float32),
        mesh=mesh,
        scratch_types=[
            pltpu.VMEM((N_CHUNKS, CHUNK), jnp.int32),
            pltpu.VMEM((NBUF, CHUNK, D), jnp.float32),
            pltpu.VMEM((NBUF, 8, 8, CHUNK), jnp.float32),
        ] + [pltpu.SemaphoreType.DMA] * (2 * NBUF),
        compiler_params=pltpu.CompilerParams(
            use_tc_tiling_on_sc=False, needs_layout_passes=False),
    )
    def gather_kernel(idx_hbm, table_hbm, out_hbm, idx_v, rows_v, trans_v,
                      *sems):
        gsem = sems[:NBUF]
        wsem = sems[NBUF:]
        wid = lax.axis_index("s") * NC + lax.axis_index("c")
        pltpu.sync_copy(idx_hbm.at[wid], idx_v)
        tc0 = wid * N_CHUNKS

        def fire(c, b):
            pltpu.async_copy(
                table_hbm.at[idx_v.at[c]], rows_v.at[b], gsem[b])

        def drain_gather(b):
            pltpu.make_async_copy(
                table_hbm.at[idx_v.at[0]], rows_v.at[b], gsem[b]).wait()

        def transpose(b):
            rows = rows_v.at[b]
            trans = trans_v.at[b]
            for t in range(8):
                for jj in range(8):
                    colv = jnp.full((16,), 8 * t + jj, jnp.int32)
                    for m in range(8):
                        seg = lax.iota(jnp.int32, 16) + (16 * m)
                        vals = plsc.load_gather(rows, [seg, colv])
                        trans[t, jj, pl.ds(16 * m, 16)] = vals

        def start_write(c, b):
            pltpu.async_copy(trans_v.at[b], out_hbm.at[:, tc0 + c], wsem[b])

        def drain_write(b):
            pltpu.make_async_copy(
                trans_v.at[b], out_hbm.at[:, tc0], wsem[b]).wait()

        for b in range(NBUF):
            fire(b, b)

        @pl.loop(0, N_ROUNDS)
        def body(r):
            for b in range(NBUF):
                c = r * NBUF + b
                drain_gather(b)

                @pl.when(r > 0)
                def _():
                    drain_write(b)

                transpose(b)
                start_write(c, b)

                @pl.when(r < N_ROUNDS - 1)
                def _():
                    fire(c + NBUF, b)

        for b in range(NBUF):
            drain_write(b)

    return gather_kernel


_gather = _make_gather()


@jax.jit
def kernel(indices, table):
    r = indices.astype(jnp.int32)
    q = r & (2 * HB - 1)
    r2 = (r & ~jnp.int32(2 * HB - 1)) + 2 * (q & (HB - 1)) + (q // HB)
    idx = r2.reshape(NW, N_CHUNKS, CHUNK)
    packed = _pack(table.T)
    flat = packed.reshape(2 * NP, D)
    out4 = _gather(idx, flat)
    out = out4.transpose(1, 3, 0, 2).reshape(B, D)
    return {"item_id": out}


# speedup vs baseline: 1.8709x; 1.8709x over previous
import functools

import jax
import jax.numpy as jnp
from jax import lax
from jax.experimental import pallas as pl
from jax.experimental.pallas import tpu as pltpu
from jax.experimental.pallas import tpu_sc as plsc

V = 1_000_000
D = 64
B = 327_680

HB = 2048
GRID = -(-V // (2 * HB))
NP = GRID * HB
MAXBLK = -(-V // HB) - 1


def _pack_kernel(a_ref, b_ref, out_ref):
    out_ref[...] = jnp.concatenate(
        [jnp.transpose(a_ref[...], (1, 0)),
         jnp.transpose(b_ref[...], (1, 0))], axis=1)


def _pack(table_t):
    return pl.pallas_call(
        _pack_kernel,
        grid=(GRID,),
        in_specs=[
            pl.BlockSpec((D, HB), lambda g: (0, jnp.minimum(2 * g, MAXBLK))),
            pl.BlockSpec((D, HB), lambda g: (0, jnp.minimum(2 * g + 1, MAXBLK))),
        ],
        out_specs=pl.BlockSpec((HB, 128), lambda g: (g, 0)),
        out_shape=jax.ShapeDtypeStruct((NP, 128), jnp.float32),
    )(table_t, table_t)


NC, NS = 2, 16
NW = NC * NS
CHUNK = 128
PER_W = B // NW
N_CHUNKS = PER_W // CHUNK
NBUF = 5
N_ROUNDS = N_CHUNKS // NBUF


TCB = B // 128


def _make_gather():
    mesh = plsc.VectorSubcoreMesh(
        core_axis_name="c", subcore_axis_name="s",
        num_cores=NC, num_subcores=NS)

    @functools.partial(
        pl.kernel,
        out_type=jax.ShapeDtypeStruct((8, TCB, 8, 128), jnp.float32),
        mesh=mesh,
        scratch_types=[
            pltpu.VMEM((N_CHUNKS, CHUNK), jnp.int32),
            pltpu.VMEM((NBUF, CHUNK, D), jnp.float32),
            pltpu.VMEM((NBUF, 8, 8, CHUNK), jnp.float32),
        ] + [pltpu.SemaphoreType.DMA] * (2 * NBUF),
        compiler_params=pltpu.CompilerParams(
            use_tc_tiling_on_sc=False, needs_layout_passes=False),
    )
    def gather_kernel(idx_hbm, table_hbm, out_hbm, idx_v, rows_v, trans_v,
                      *sems):
        gsem = sems[:NBUF]
        wsem = sems[NBUF:]
        wid = lax.axis_index("s") * NC + lax.axis_index("c")
        pltpu.sync_copy(idx_hbm.at[wid], idx_v)
        tc0 = wid * N_CHUNKS

        def fire(c, b):
            pltpu.async_copy(
                table_hbm.at[idx_v.at[c]], rows_v.at[b], gsem[b])

        def drain_gather(b):
            pltpu.make_async_copy(
                table_hbm.at[idx_v.at[0]], rows_v.at[b], gsem[b]).wait()

        iota = lax.iota(jnp.int32, 16)

        def transpose(b):
            rows = rows_v.at[b]
            trans = trans_v.at[b]

            @pl.loop(0, 64)
            def diag(j0):
                colv = (j0 + iota) & 63
                tv = colv >> 3
                jv = colv & 7
                for m in range(8):
                    rowv = iota + 16 * m
                    vals = plsc.load_gather(rows, [rowv, colv])
                    plsc.store_scatter(trans, [tv, jv, rowv], vals)

        def start_write(c, b):
            pltpu.async_copy(trans_v.at[b], out_hbm.at[:, tc0 + c], wsem[b])

        def drain_write(b):
            pltpu.make_async_copy(
                trans_v.at[b], out_hbm.at[:, tc0], wsem[b]).wait()

        for b in range(NBUF):
            fire(b, b)

        @pl.loop(0, N_ROUNDS)
        def body(r):
            for b in range(NBUF):
                c = r * NBUF + b
                drain_gather(b)

                @pl.when(r > 0)
                def _():
                    drain_write(b)

                transpose(b)
                start_write(c, b)

                @pl.when(r < N_ROUNDS - 1)
                def _():
                    fire(c + NBUF, b)

        for b in range(NBUF):
            drain_write(b)

    return gather_kernel


_gather = _make_gather()


@jax.jit
def kernel(indices, table):
    r = indices.astype(jnp.int32)
    q = r & (2 * HB - 1)
    r2 = (r & ~jnp.int32(2 * HB - 1)) + 2 * (q & (HB - 1)) + (q // HB)
    idx = r2.reshape(NW, N_CHUNKS, CHUNK)
    packed = _pack(table.T)
    flat = packed.reshape(2 * NP, D)
    out4 = _gather(idx, flat)
    out = out4.transpose(1, 3, 0, 2).reshape(B, D)
    return {"item_id": out}


# speedup vs baseline: 2.1553x; 1.1520x over previous
import functools

import jax
import jax.numpy as jnp
from jax import lax
from jax.experimental import pallas as pl
from jax.experimental.pallas import tpu as pltpu
from jax.experimental.pallas import tpu_sc as plsc

V = 1_000_000
D = 64
B = 327_680

HB = 4096
GRID = -(-V // (2 * HB))
NP = GRID * HB
MAXBLK = -(-V // HB) - 1


def _pack_kernel(a_ref, b_ref, out_ref):
    out_ref[...] = jnp.concatenate(
        [jnp.transpose(a_ref[...], (1, 0)),
         jnp.transpose(b_ref[...], (1, 0))], axis=1)


def _pack(table_t):
    return pl.pallas_call(
        _pack_kernel,
        grid=(GRID,),
        in_specs=[
            pl.BlockSpec((D, HB), lambda g: (0, jnp.minimum(2 * g, MAXBLK))),
            pl.BlockSpec((D, HB), lambda g: (0, jnp.minimum(2 * g + 1, MAXBLK))),
        ],
        out_specs=pl.BlockSpec((HB, 128), lambda g: (g, 0)),
        out_shape=jax.ShapeDtypeStruct((NP, 128), jnp.float32),
    )(table_t, table_t)


NC, NS = 2, 16
NW = NC * NS
CHUNK = 128
PER_W = B // NW
N_CHUNKS = PER_W // CHUNK
NBUF = 5
N_ROUNDS = N_CHUNKS // NBUF


TCB = B // 128


def _make_gather():
    mesh = plsc.VectorSubcoreMesh(
        core_axis_name="c", subcore_axis_name="s",
        num_cores=NC, num_subcores=NS)

    @functools.partial(
        pl.kernel,
        out_type=jax.ShapeDtypeStruct((8, TCB, 8, 128), jnp.float32),
        mesh=mesh,
        scratch_types=[
            pltpu.VMEM((N_CHUNKS, CHUNK), jnp.int32),
            pltpu.VMEM((NBUF, CHUNK, D), jnp.float32),
            pltpu.VMEM((NBUF, 8, 8, CHUNK), jnp.float32),
        ] + [pltpu.SemaphoreType.DMA] * (2 * NBUF),
        compiler_params=pltpu.CompilerParams(
            use_tc_tiling_on_sc=False, needs_layout_passes=False),
    )
    def gather_kernel(idx_hbm, table_hbm, out_hbm, idx_v, rows_v, trans_v,
                      *sems):
        gsem = sems[:NBUF]
        wsem = sems[NBUF:]
        wid = lax.axis_index("s") * NC + lax.axis_index("c")
        pltpu.sync_copy(idx_hbm.at[wid], idx_v)
        tc0 = wid * N_CHUNKS

        def fire(c, b):
            pltpu.async_copy(
                table_hbm.at[idx_v.at[c]], rows_v.at[b], gsem[b])

        def drain_gather(b):
            pltpu.make_async_copy(
                table_hbm.at[idx_v.at[0]], rows_v.at[b], gsem[b]).wait()

        iota = lax.iota(jnp.int32, 16)

        def transpose(b):
            rows = rows_v.at[b]
            trans = trans_v.at[b]

            @pl.loop(0, 64)
            def diag(j0):
                colv = (j0 + iota) & 63
                tv = colv >> 3
                jv = colv & 7
                for m in range(8):
                    rowv = iota + 16 * m
                    vals = plsc.load_gather(rows, [rowv, colv])
                    plsc.store_scatter(trans, [tv, jv, rowv], vals)

        def start_write(c, b):
            pltpu.async_copy(trans_v.at[b], out_hbm.at[:, tc0 + c], wsem[b])

        def drain_write(b):
            pltpu.make_async_copy(
                trans_v.at[b], out_hbm.at[:, tc0], wsem[b]).wait()

        for b in range(NBUF):
            fire(b, b)

        @pl.loop(0, N_ROUNDS)
        def body(r):
            for b in range(NBUF):
                c = r * NBUF + b
                drain_gather(b)

                @pl.when(r > 0)
                def _():
                    drain_write(b)

                transpose(b)
                start_write(c, b)

                @pl.when(r < N_ROUNDS - 1)
                def _():
                    fire(c + NBUF, b)

        for b in range(NBUF):
            drain_write(b)

    return gather_kernel


_gather = _make_gather()


@jax.jit
def kernel(indices, table):
    r = indices.astype(jnp.int32)
    q = r & (2 * HB - 1)
    r2 = (r & ~jnp.int32(2 * HB - 1)) + 2 * (q & (HB - 1)) + (q // HB)
    idx = r2.reshape(NW, N_CHUNKS, CHUNK)
    packed = _pack(table.T)
    flat = packed.reshape(2 * NP, D)
    out4 = _gather(idx, flat)
    out = out4.transpose(1, 3, 0, 2).reshape(B, D)
    return {"item_id": out}
